# Initial kernel scaffold; baseline (speedup 1.0000x reference)
import jax
import jax.numpy as jnp
from jax.experimental import pallas as pl


def kernel(shape_id, colour_id, pos_id, edge_index, batch, shape_emb, col_emb, pos_emb, Wl1, bl1, Wr1, Wl2, bl2, Wr2, Wlin, blin):
    raise NotImplementedError("write your pallas kernel here")


# speedup vs baseline: 4.3551x; 4.3551x over previous
import functools

import jax
import jax.numpy as jnp
from jax import lax
from jax.experimental import pallas as pl
from jax.experimental.pallas import tpu as pltpu
from jax.experimental.pallas import tpu_sc as plsc

N = 10000
E = 320000
D = 128
NPAD = 10240
NC = 2
NS = 16
NW = NC * NS
L = 16
GRAPHS = 64

EP_CHUNKS = 79
EPT = EP_CHUNKS * 128
EPAD = EPT * NW
ROWS_PER_TILE = NPAD // NW
ROWS_PER_SUB = NPAD // NS

_mesh = plsc.VectorSubcoreMesh(core_axis_name="c", subcore_axis_name="s")


@functools.partial(
    pl.kernel,
    out_type=jax.ShapeDtypeStruct((NPAD, D), jnp.float32),
    mesh=_mesh,
    scratch_types=[
        pltpu.VMEM((5, 64), jnp.int32),
        pltpu.VMEM((5, 64), jnp.int32),
        pltpu.VMEM((5, 64), jnp.int32),
        pltpu.VMEM((64, D), jnp.float32),
        pltpu.VMEM((64, D), jnp.float32),
        pltpu.VMEM((64, D), jnp.float32),
        pltpu.SemaphoreType.DMA,
    ],
)
def _sc_embed(sid2, cid2, pid2, semb, cemb, pemb, x_out,
              si_v, ci_v, pi_v, a_v, b_v, c_v, sem):
    cid = lax.axis_index("c")
    sid = lax.axis_index("s")
    wid = sid * NC + cid
    nb = wid * ROWS_PER_TILE
    pltpu.sync_copy(sid2.at[wid], si_v)
    pltpu.sync_copy(cid2.at[wid], ci_v)
    pltpu.sync_copy(pid2.at[wid], pi_v)

    @pl.loop(0, 5)
    def _chunk(k):
        cp1 = pltpu.async_copy(semb.at[si_v.at[k]], a_v, sem)
        cp2 = pltpu.async_copy(cemb.at[ci_v.at[k]], b_v, sem)
        cp3 = pltpu.async_copy(pemb.at[pi_v.at[k]], c_v, sem)
        cp1.wait()
        cp2.wait()
        cp3.wait()

        @pl.loop(0, 64)
        def _row(r):
            for c8 in range(8):
                sl = pl.ds(c8 * L, L)
                a_v[r, sl] = a_v[r, sl] + b_v[r, sl] + c_v[r, sl]

        pltpu.sync_copy(a_v, x_out.at[pl.ds(nb + k * 64, 64)])


def _make_sc_agg(with_cnt):
    outs = [jax.ShapeDtypeStruct((NC, NPAD, D), jnp.float32)]
    scratch = [
        pltpu.VMEM((EP_CHUNKS, 128), jnp.int32),
        pltpu.VMEM((EP_CHUNKS, 128), jnp.int32),
        pltpu.VMEM((128, D), jnp.float32),
        pltpu.VMEM_SHARED((NPAD, D), jnp.float32),
        pltpu.SemaphoreType.DMA,
    ]
    if with_cnt:
        outs.append(jax.ShapeDtypeStruct((NC, NPAD), jnp.float32))
        scratch += [
            pltpu.VMEM((128,), jnp.float32),
            pltpu.VMEM((ROWS_PER_SUB,), jnp.float32),
            pltpu.VMEM_SHARED((NPAD,), jnp.float32),
        ]

    def body(x_hbm, src2, dst2, *rest):
        if with_cnt:
            (agg_out, cnt_out, src_v, dst_v, rows_v, agg_sh, sem,
             ones_v, zc_v, cnt_sh) = rest
        else:
            agg_out, src_v, dst_v, rows_v, agg_sh, sem = rest
        cid = lax.axis_index("c")
        sid = lax.axis_index("s")
        wid = sid * NC + cid
        zb = sid * ROWS_PER_SUB

        @pl.loop(0, 128)
        def _z(r):
            for c8 in range(8):
                rows_v[r, pl.ds(c8 * L, L)] = jnp.zeros((L,), jnp.float32)

        for i in range(ROWS_PER_SUB // 128):
            pltpu.sync_copy(rows_v, agg_sh.at[pl.ds(zb + i * 128, 128)])
        if with_cnt:
            @pl.loop(0, ROWS_PER_SUB // L)
            def _zc(r):
                zc_v[pl.ds(r * L, L)] = jnp.zeros((L,), jnp.float32)

            pltpu.sync_copy(zc_v, cnt_sh.at[pl.ds(zb, ROWS_PER_SUB)])

            @pl.loop(0, 128 // L)
            def _o(r):
                ones_v[pl.ds(r * L, L)] = jnp.ones((L,), jnp.float32)

        pltpu.sync_copy(src2.at[wid], src_v)
        pltpu.sync_copy(dst2.at[wid], dst_v)

        plsc.subcore_barrier()

        @pl.loop(0, EP_CHUNKS)
        def _edge(j):
            pltpu.async_copy(x_hbm.at[src_v.at[j]], rows_v, sem).wait()
            pltpu.sync_copy(rows_v, agg_sh.at[dst_v.at[j]], add=True)
            if with_cnt:
                pltpu.sync_copy(ones_v, cnt_sh.at[dst_v.at[j]], add=True)

        plsc.subcore_barrier()
        pltpu.sync_copy(agg_sh.at[pl.ds(zb, ROWS_PER_SUB)],
                        agg_out.at[cid, pl.ds(zb, ROWS_PER_SUB)])
        if with_cnt:
            pltpu.sync_copy(cnt_sh.at[pl.ds(zb, ROWS_PER_SUB)],
                            cnt_out.at[cid, pl.ds(zb, ROWS_PER_SUB)])

    return pl.kernel(body, out_type=outs, mesh=_mesh, scratch_types=scratch)


_sc_agg_cnt = _make_sc_agg(True)
_sc_agg = _make_sc_agg(False)

BLK = 512
GRID = NPAD // BLK


def _tc_sage_body(a0_ref, a1_ref, c0_ref, c1_ref, x_ref, wl_ref, wr_ref,
                  b_ref, o_ref):
    c = jnp.maximum(c0_ref[...] + c1_ref[...], 1.0)
    a = (a0_ref[...] + a1_ref[...]) / c
    h = (jnp.dot(a, wl_ref[...], preferred_element_type=jnp.float32)
         + jnp.dot(x_ref[...], wr_ref[...], preferred_element_type=jnp.float32)
         + b_ref[...])
    o_ref[...] = jnp.maximum(h, 0.0)


def _tc_sage(a0, a1, c0, c1, x, wlT, wrT, b2d):
    return pl.pallas_call(
        _tc_sage_body,
        grid=(GRID,),
        in_specs=[
            pl.BlockSpec((BLK, D), lambda i: (i, 0)),
            pl.BlockSpec((BLK, D), lambda i: (i, 0)),
            pl.BlockSpec((BLK, 1), lambda i: (i, 0)),
            pl.BlockSpec((BLK, 1), lambda i: (i, 0)),
            pl.BlockSpec((BLK, D), lambda i: (i, 0)),
            pl.BlockSpec((D, D), lambda i: (0, 0)),
            pl.BlockSpec((D, D), lambda i: (0, 0)),
            pl.BlockSpec((1, D), lambda i: (0, 0)),
        ],
        out_specs=pl.BlockSpec((BLK, D), lambda i: (i, 0)),
        out_shape=jax.ShapeDtypeStruct((NPAD, D), jnp.float32),
    )(a0, a1, c0, c1, x, wlT, wrT, b2d)


def _tc_pool_body(h_ref, b_ref, wlin_ref, blin_ref, o_ref, g_ref):
    i = pl.program_id(0)

    @pl.when(i == 0)
    def _():
        g_ref[...] = jnp.zeros_like(g_ref)

    bvals = b_ref[...]
    iot = lax.broadcasted_iota(jnp.int32, (BLK, GRAPHS), 1)
    oh = (iot == bvals).astype(jnp.float32)
    g_ref[...] += lax.dot_general(
        oh, h_ref[...], (((0,), (0,)), ((), ())),
        preferred_element_type=jnp.float32)

    @pl.when(i == GRID - 1)
    def _():
        o_ref[...] = (jnp.dot(g_ref[...], wlin_ref[...],
                              preferred_element_type=jnp.float32)
                      + blin_ref[...])


def _tc_pool(h2, bat2, wlinT, blin8):
    return pl.pallas_call(
        _tc_pool_body,
        grid=(GRID,),
        in_specs=[
            pl.BlockSpec((BLK, D), lambda i: (i, 0)),
            pl.BlockSpec((BLK, 1), lambda i: (i, 0)),
            pl.BlockSpec((D, 8), lambda i: (0, 0)),
            pl.BlockSpec((1, 8), lambda i: (0, 0)),
        ],
        out_specs=pl.BlockSpec((GRAPHS, 8), lambda i: (0, 0)),
        out_shape=jax.ShapeDtypeStruct((GRAPHS, 8), jnp.float32),
        scratch_shapes=[pltpu.VMEM((GRAPHS, D), jnp.float32)],
    )(h2, bat2, wlinT, blin8)


def kernel(shape_id, colour_id, pos_id, edge_index, batch,
           shape_emb, col_emb, pos_emb,
           Wl1, bl1, Wr1, Wl2, bl2, Wr2, Wlin, blin):
    pad = NPAD - N
    sid2 = jnp.pad(shape_id.astype(jnp.int32), (0, pad)).reshape(NW, 5, 64)
    cid2 = jnp.pad(colour_id.astype(jnp.int32), (0, pad)).reshape(NW, 5, 64)
    pid2 = jnp.pad(pos_id.astype(jnp.int32), (0, pad)).reshape(NW, 5, 64)
    src2 = jnp.pad(edge_index[0].astype(jnp.int32),
                   (0, EPAD - E)).reshape(NW, EP_CHUNKS, 128)
    dst2 = jnp.pad(edge_index[1].astype(jnp.int32), (0, EPAD - E),
                   constant_values=N).reshape(NW, EP_CHUNKS, 128)
    bat2 = jnp.pad(batch.astype(jnp.int32), (0, pad),
                   constant_values=GRAPHS).reshape(NPAD, 1)

    x = _sc_embed(sid2, cid2, pid2, shape_emb, col_emb, pos_emb)
    aggp, cntp = _sc_agg_cnt(x, src2, dst2)
    c0 = cntp[0].reshape(NPAD, 1)
    c1 = cntp[1].reshape(NPAD, 1)
    h1 = _tc_sage(aggp[0], aggp[1], c0, c1, x,
                  Wl1.T, Wr1.T, bl1.reshape(1, D))
    aggp2 = _sc_agg(h1, src2, dst2)[0]
    h2 = _tc_sage(aggp2[0], aggp2[1], c0, c1, h1,
                  Wl2.T, Wr2.T, bl2.reshape(1, D))
    wlinT = jnp.zeros((D, 8), jnp.float32).at[:, :2].set(Wlin.T)
    blin8 = jnp.zeros((1, 8), jnp.float32).at[0, :2].set(blin)
    outp = _tc_pool(h2, bat2, wlinT, blin8)
    return outp[:, :2]
